# baseline (speedup 1.0000x reference)
import functools

import jax
import jax.numpy as jnp
from jax import lax
from jax.experimental import pallas as pl
from jax.experimental.pallas import tpu as pltpu
from jax.experimental.pallas import tpu_sc as plsc

N = 10000
E = 320000
D = 128
NC = 2
NS = 16
NW = NC * NS
EPT = E // NW
CH = 40
NCH = EPT // CH
RPT = N // NS
DW = 8
NB = 5

_MESH = plsc.VectorSubcoreMesh(core_axis_name="c", subcore_axis_name="s")
_SC_PARAMS = pltpu.CompilerParams(use_tc_tiling_on_sc=False)


def _make_agg(with_deg):
  out_type = jax.ShapeDtypeStruct((NC, N, D), jnp.float32)
  if with_deg:
    out_type = [out_type, jax.ShapeDtypeStruct((NC, N, DW), jnp.float32)]
  scratch = [
      pltpu.VMEM((NCH, CH), jnp.int32),
      pltpu.VMEM((NCH, CH), jnp.int32),
      [pltpu.VMEM((CH, D), jnp.float32) for _ in range(NB)],
      [pltpu.SemaphoreType.DMA for _ in range(NB)],
      [pltpu.SemaphoreType.DMA for _ in range(NB)],
      pltpu.VMEM_SHARED((N, D), jnp.float32),
  ]
  if with_deg:
    scratch += [
        pltpu.VMEM((CH, DW), jnp.float32),
        pltpu.VMEM_SHARED((N, DW), jnp.float32),
        pltpu.SemaphoreType.DMA,
    ]

  def agg_body(*args):
    if with_deg:
      (x_hbm, src_hbm, dst_hbm, cst_hbm, out_hbm, deg_hbm, src_v, dst_v,
       bufs, gsem, ssem, acc_sp, obuf, deg_sp, dsem) = args
    else:
      (x_hbm, src_hbm, dst_hbm, out_hbm, src_v, dst_v, bufs, gsem, ssem,
       acc_sp) = args
    c = lax.axis_index("c")
    s = lax.axis_index("s")
    wid = c * NS + s
    pltpu.sync_copy(src_hbm.at[wid], src_v)
    pltpu.sync_copy(dst_hbm.at[wid], dst_v)

    base = s * RPT
    nfull = RPT // CH
    rem = RPT - nfull * CH

    for u in range(NB - 1):
      pltpu.async_copy(x_hbm.at[src_v.at[u]], bufs[u], gsem[u])

    zb = bufs[NB - 1]
    vec = jnp.zeros((16,), jnp.float32)

    def frow(r, carry):
      for jj in range(D // 16):
        zb[r, pl.ds(jj * 16, 16)] = vec
      return carry

    lax.fori_loop(0, CH, frow, 0)

    def zcp(k, carry):
      pltpu.sync_copy(zb, acc_sp.at[pl.ds(base + k * CH, CH)])
      return carry

    lax.fori_loop(0, nfull, zcp, 0)
    if rem:
      pltpu.sync_copy(zb.at[pl.ds(0, rem)],
                      acc_sp.at[pl.ds(base + nfull * CH, rem)])
    if with_deg:
      pltpu.sync_copy(cst_hbm.at[0], obuf)

      def dzcp(k, carry):
        pltpu.sync_copy(obuf, deg_sp.at[pl.ds(base + k * CH, CH)])
        return carry

      lax.fori_loop(0, nfull, dzcp, 0)
      if rem:
        pltpu.sync_copy(obuf.at[pl.ds(0, rem)],
                        deg_sp.at[pl.ds(base + nfull * CH, rem)])
      pltpu.sync_copy(cst_hbm.at[1], obuf)
    plsc.subcore_barrier()

    def body(i, carry):
      j0 = i * NB
      for u in range(NB):
        j = j0 + u
        v = (u + NB - 1) % NB
        pltpu.make_async_copy(x_hbm.at[src_v.at[j]], bufs[u],
                              gsem[u]).wait()
        pltpu.async_copy(bufs[u], acc_sp.at[dst_v.at[j]], ssem[u],
                         add=True)
        if with_deg:
          pltpu.async_copy(obuf, deg_sp.at[dst_v.at[j]], dsem, add=True)
        if u == 0:
          @pl.when(i > 0)
          def _(j=j, v=v):
            pltpu.make_async_copy(bufs[v], acc_sp.at[dst_v.at[j - 1]],
                                  ssem[v]).wait()
        else:
          pltpu.make_async_copy(bufs[v], acc_sp.at[dst_v.at[j - 1]],
                                ssem[v]).wait()

        @pl.when(j + NB - 1 < NCH)
        def _(j=j, v=v):
          pltpu.async_copy(x_hbm.at[src_v.at[j + NB - 1]], bufs[v], gsem[v])
      return carry

    lax.fori_loop(0, NCH // NB, body, 0)
    pltpu.make_async_copy(bufs[NB - 1], acc_sp.at[dst_v.at[NCH - 1]],
                          ssem[NB - 1]).wait()
    if with_deg:
      def ddrain(j, carry):
        pltpu.make_async_copy(obuf, deg_sp.at[dst_v.at[j]], dsem).wait()
        return carry

      lax.fori_loop(0, NCH, ddrain, 0)
    plsc.subcore_barrier()
    pltpu.sync_copy(acc_sp.at[pl.ds(base, RPT)],
                    out_hbm.at[c, pl.ds(base, RPT)])
    if with_deg:
      pltpu.sync_copy(deg_sp.at[pl.ds(base, RPT)],
                      deg_hbm.at[c, pl.ds(base, RPT)])

  return pl.kernel(agg_body, mesh=_MESH, compiler_params=_SC_PARAMS,
                   out_type=out_type, scratch_types=scratch)


_agg_deg = _make_agg(True)
_agg = _make_agg(False)


RB = 5000


def _dense_body(agg_ref, degp_ref, x_ref, wl_ref, bl_ref, wr_ref, o_ref):
  a = agg_ref[0] + agg_ref[1]
  deg = degp_ref[0, :, 0:1] + degp_ref[1, :, 0:1]
  recip = 1.0 / jnp.maximum(deg, 1.0)
  mean = a * recip
  y = lax.dot_general(mean, wl_ref[...], (((1,), (1,)), ((), ())),
                      preferred_element_type=jnp.float32)
  y = y + lax.dot_general(x_ref[...], wr_ref[...], (((1,), (1,)), ((), ())),
                          preferred_element_type=jnp.float32)
  y = y + bl_ref[...][None, :]
  o_ref[...] = jnp.where(y >= 0, y, 0.01 * y)


_dense = pl.pallas_call(
    _dense_body,
    grid=(N // RB,),
    in_specs=[
        pl.BlockSpec((NC, RB, D), lambda i: (0, i, 0)),
        pl.BlockSpec((NC, RB, DW), lambda i: (0, i, 0)),
        pl.BlockSpec((RB, D), lambda i: (i, 0)),
        pl.BlockSpec((D, D), lambda i: (0, 0)),
        pl.BlockSpec((D,), lambda i: (0,)),
        pl.BlockSpec((D, D), lambda i: (0, 0)),
    ],
    out_specs=pl.BlockSpec((RB, D), lambda i: (i, 0)),
    out_shape=jax.ShapeDtypeStruct((N, D), jnp.float32),
)


def kernel(x, edge_index, W1l, b1l, W1r, W2l, b2l, W2r):
  src = edge_index[0].reshape(NW, NCH, CH)
  dst = edge_index[1].reshape(NW, NCH, CH)
  cst = jnp.stack([jnp.zeros((CH, DW), jnp.float32),
                   jnp.ones((CH, DW), jnp.float32)])
  agg1, degp = _agg_deg(x, src, dst, cst)
  h = _dense(agg1, degp, x, W1l, b1l, W1r)
  agg2 = _agg(h, src, dst)
  return _dense(agg2, degp, h, W2l, b2l, W2r)


# speedup vs baseline: 1.0021x; 1.0021x over previous
import functools

import jax
import jax.numpy as jnp
from jax import lax
from jax.experimental import pallas as pl
from jax.experimental.pallas import tpu as pltpu
from jax.experimental.pallas import tpu_sc as plsc

N = 10000
E = 320000
D = 128
NC = 2
NS = 16
NW = NC * NS
EPT = E // NW
CH = 40
NCH = EPT // CH
RPT = N // NS
DW = 8
NB = 5

_MESH = plsc.VectorSubcoreMesh(core_axis_name="c", subcore_axis_name="s")
_SC_PARAMS = pltpu.CompilerParams(use_tc_tiling_on_sc=False)


def _make_agg(with_deg):
  out_type = jax.ShapeDtypeStruct((NC, N, D), jnp.float32)
  if with_deg:
    out_type = [out_type, jax.ShapeDtypeStruct((NC, N, DW), jnp.float32)]
  scratch = [
      pltpu.VMEM((NCH, CH), jnp.int32),
      pltpu.VMEM((NCH, CH), jnp.int32),
      [pltpu.VMEM((CH, D), jnp.float32) for _ in range(NB)],
      [pltpu.SemaphoreType.DMA for _ in range(NB)],
      [pltpu.SemaphoreType.DMA for _ in range(NB)],
      pltpu.VMEM_SHARED((N, D), jnp.float32),
  ]
  if with_deg:
    scratch += [
        pltpu.VMEM((CH, DW), jnp.float32),
        pltpu.VMEM_SHARED((N, DW), jnp.float32),
        pltpu.SemaphoreType.DMA,
    ]

  def agg_body(*args):
    if with_deg:
      (x_hbm, src_hbm, dst_hbm, cst_hbm, out_hbm, deg_hbm, src_v, dst_v,
       bufs, gsem, ssem, acc_sp, obuf, deg_sp, dsem) = args
    else:
      (x_hbm, src_hbm, dst_hbm, out_hbm, src_v, dst_v, bufs, gsem, ssem,
       acc_sp) = args
    c = lax.axis_index("c")
    s = lax.axis_index("s")
    wid = c * NS + s
    pltpu.sync_copy(src_hbm.at[wid], src_v)
    pltpu.sync_copy(dst_hbm.at[wid], dst_v)

    base = s * RPT
    nfull = RPT // CH
    rem = RPT - nfull * CH

    for u in range(NB - 1):
      pltpu.async_copy(x_hbm.at[src_v.at[u]], bufs[u], gsem[u])

    zb = bufs[NB - 1]
    vec = jnp.zeros((16,), jnp.float32)

    def frow(r, carry):
      for jj in range(D // 16):
        zb[r, pl.ds(jj * 16, 16)] = vec
      return carry

    lax.fori_loop(0, CH, frow, 0)

    def zcp(k, carry):
      pltpu.sync_copy(zb, acc_sp.at[pl.ds(base + k * CH, CH)])
      return carry

    lax.fori_loop(0, nfull, zcp, 0)
    if rem:
      pltpu.sync_copy(zb.at[pl.ds(0, rem)],
                      acc_sp.at[pl.ds(base + nfull * CH, rem)])
    if with_deg:
      pltpu.sync_copy(cst_hbm.at[0], obuf)

      def dzcp(k, carry):
        pltpu.sync_copy(obuf, deg_sp.at[pl.ds(base + k * CH, CH)])
        return carry

      lax.fori_loop(0, nfull, dzcp, 0)
      if rem:
        pltpu.sync_copy(obuf.at[pl.ds(0, rem)],
                        deg_sp.at[pl.ds(base + nfull * CH, rem)])
      pltpu.sync_copy(cst_hbm.at[1], obuf)
    plsc.subcore_barrier()

    def body(i, carry):
      j0 = i * NB
      for u in range(NB):
        j = j0 + u
        v = (u + NB - 1) % NB
        pltpu.make_async_copy(x_hbm.at[src_v.at[j]], bufs[u],
                              gsem[u]).wait()
        pltpu.async_copy(bufs[u], acc_sp.at[dst_v.at[j]], ssem[u],
                         add=True)
        if with_deg:
          pltpu.async_copy(obuf, deg_sp.at[dst_v.at[j]], dsem, add=True)
        if u == 0:
          @pl.when(i > 0)
          def _(j=j, v=v):
            pltpu.make_async_copy(bufs[v], acc_sp.at[dst_v.at[j - 1]],
                                  ssem[v]).wait()
        else:
          pltpu.make_async_copy(bufs[v], acc_sp.at[dst_v.at[j - 1]],
                                ssem[v]).wait()

        @pl.when(j + NB - 1 < NCH)
        def _(j=j, v=v):
          pltpu.async_copy(x_hbm.at[src_v.at[j + NB - 1]], bufs[v], gsem[v])
      return carry

    lax.fori_loop(0, NCH // NB, body, 0)
    pltpu.make_async_copy(bufs[NB - 1], acc_sp.at[dst_v.at[NCH - 1]],
                          ssem[NB - 1]).wait()
    if with_deg:
      def ddrain(j, carry):
        pltpu.make_async_copy(obuf, deg_sp.at[dst_v.at[j]], dsem).wait()
        return carry

      lax.fori_loop(0, NCH, ddrain, 0)
    plsc.subcore_barrier()
    pltpu.sync_copy(acc_sp.at[pl.ds(base, RPT)],
                    out_hbm.at[c, pl.ds(base, RPT)])
    if with_deg:
      pltpu.sync_copy(deg_sp.at[pl.ds(base, RPT)],
                      deg_hbm.at[c, pl.ds(base, RPT)])

  return pl.kernel(agg_body, mesh=_MESH, compiler_params=_SC_PARAMS,
                   out_type=out_type, scratch_types=scratch)


_agg_deg = _make_agg(True)
_agg = _make_agg(False)


RB = 2000


def _dense_body(agg_ref, degp_ref, x_ref, wl_ref, bl_ref, wr_ref, o_ref):
  a = agg_ref[0] + agg_ref[1]
  deg = degp_ref[0, :, 0:1] + degp_ref[1, :, 0:1]
  recip = 1.0 / jnp.maximum(deg, 1.0)
  mean = a * recip
  y = lax.dot_general(mean, wl_ref[...], (((1,), (1,)), ((), ())),
                      preferred_element_type=jnp.float32)
  y = y + lax.dot_general(x_ref[...], wr_ref[...], (((1,), (1,)), ((), ())),
                          preferred_element_type=jnp.float32)
  y = y + bl_ref[...][None, :]
  o_ref[...] = jnp.where(y >= 0, y, 0.01 * y)


_dense = pl.pallas_call(
    _dense_body,
    grid=(N // RB,),
    in_specs=[
        pl.BlockSpec((NC, RB, D), lambda i: (0, i, 0)),
        pl.BlockSpec((NC, RB, DW), lambda i: (0, i, 0)),
        pl.BlockSpec((RB, D), lambda i: (i, 0)),
        pl.BlockSpec((D, D), lambda i: (0, 0)),
        pl.BlockSpec((D,), lambda i: (0,)),
        pl.BlockSpec((D, D), lambda i: (0, 0)),
    ],
    out_specs=pl.BlockSpec((RB, D), lambda i: (i, 0)),
    out_shape=jax.ShapeDtypeStruct((N, D), jnp.float32),
)


def kernel(x, edge_index, W1l, b1l, W1r, W2l, b2l, W2r):
  src = edge_index[0].reshape(NW, NCH, CH)
  dst = edge_index[1].reshape(NW, NCH, CH)
  cst = jnp.stack([jnp.zeros((CH, DW), jnp.float32),
                   jnp.ones((CH, DW), jnp.float32)])
  agg1, degp = _agg_deg(x, src, dst, cst)
  h = _dense(agg1, degp, x, W1l, b1l, W1r)
  agg2 = _agg(h, src, dst)
  return _dense(agg2, degp, h, W2l, b2l, W2r)
